# baseline (speedup 1.0000x reference)
import functools

import jax
import jax.numpy as jnp
from jax import lax
from jax.experimental import pallas as pl
from jax.experimental.pallas import tpu as pltpu
from jax.experimental.pallas import tpu_sc as plsc

VOCAB = 1000000
HIDDEN = 64
OUT = 2
B = 4096
L = 200

NC = 2
NS = 16
NW = NC * NS
SPW = B // NW
HALF = L // 2
LANES = 16
OUTP = 128


def _sc_body(text_ref, table_ref, out_ref, idx_v, rows_v, sums_v, sem0, sem1):
    c = lax.axis_index("c")
    s = lax.axis_index("s")
    wid = s * NC + c
    base2 = wid * (2 * SPW)

    pltpu.sync_copy(text_ref.at[pl.ds(base2, 2 * SPW)], idx_v)

    sems = (sem0, sem1)

    def fire(sample, parity):
        r0 = 2 * sample
        pltpu.async_copy(table_ref.at[idx_v.at[r0]],
                         rows_v.at[parity].at[pl.ds(0, HALF)], sems[parity])
        pltpu.async_copy(table_ref.at[idx_v.at[r0 + 1]],
                         rows_v.at[parity].at[pl.ds(HALF, HALF)], sems[parity])

    def drain(parity):
        pltpu.make_async_copy(table_ref.at[idx_v.at[0]],
                              rows_v.at[parity].at[pl.ds(0, HALF)],
                              sems[parity]).wait()
        pltpu.make_async_copy(table_ref.at[idx_v.at[0]],
                              rows_v.at[parity].at[pl.ds(HALF, HALF)],
                              sems[parity]).wait()

    fire(0, 0)
    fire(1, 1)

    zero = jnp.zeros((LANES,), jnp.float32)

    def pair_body(p, carry):
        for parity in range(2):
            sample = 2 * p + parity
            drain(parity)

            def acc_body(r, acc):
                a0, a1, a2, a3 = acc
                a0 = a0 + rows_v[parity, r, pl.ds(0, LANES)]
                a1 = a1 + rows_v[parity, r, pl.ds(LANES, LANES)]
                a2 = a2 + rows_v[parity, r, pl.ds(2 * LANES, LANES)]
                a3 = a3 + rows_v[parity, r, pl.ds(3 * LANES, LANES)]
                return a0, a1, a2, a3

            a0, a1, a2, a3 = lax.fori_loop(
                0, L, acc_body, (zero, zero, zero, zero), unroll=8)

            nxt = sample + 2

            @pl.when(nxt < SPW)
            def _():
                fire(nxt, parity)

            sums_v[sample, pl.ds(0, LANES)] = a0
            sums_v[sample, pl.ds(LANES, LANES)] = a1
            sums_v[sample, pl.ds(2 * LANES, LANES)] = a2
            sums_v[sample, pl.ds(3 * LANES, LANES)] = a3
        return carry

    lax.fori_loop(0, SPW // 2, pair_body, 0)

    pltpu.sync_copy(sums_v, out_ref.at[pl.ds(wid * SPW, SPW)])


@functools.partial(
    pl.kernel,
    out_type=jax.ShapeDtypeStruct((B, HIDDEN), jnp.float32),
    mesh=plsc.VectorSubcoreMesh(core_axis_name="c", subcore_axis_name="s",
                                num_cores=NC, num_subcores=NS),
    scratch_types=[
        pltpu.VMEM((2 * SPW, HALF), jnp.int32),
        pltpu.VMEM((2, L, HIDDEN), jnp.float32),
        pltpu.VMEM((SPW, HIDDEN), jnp.float32),
        pltpu.SemaphoreType.DMA,
        pltpu.SemaphoreType.DMA,
    ],
    compiler_params=pltpu.CompilerParams(use_tc_tiling_on_sc=False),
)
def _sc_sums(text_ref, table_ref, out_ref, idx_v, rows_v, sums_v, sem0, sem1):
    _sc_body(text_ref, table_ref, out_ref, idx_v, rows_v, sums_v, sem0, sem1)


def _tc_linear_body(sums_ref, w_ref, b_ref, o_ref):
    o_ref[...] = (jnp.dot(sums_ref[...], w_ref[...],
                          preferred_element_type=jnp.float32) * (1.0 / L)
                  + b_ref[...][None, :])


_tc_linear = pl.pallas_call(
    _tc_linear_body,
    out_shape=jax.ShapeDtypeStruct((B, OUTP), jnp.float32),
)


def kernel(text, emb_table, fc1_w, fc1_b):
    text2 = text.astype(jnp.int32).reshape(2 * B, HALF)
    sums = _sc_sums(text2, emb_table)
    w_pad = jnp.zeros((HIDDEN, OUTP), jnp.float32).at[:, :OUT].set(fc1_w.T)
    b_pad = jnp.zeros((OUTP,), jnp.float32).at[:OUT].set(fc1_b)
    out = _tc_linear(sums, w_pad, b_pad)
    return out[:, :OUT]


# speedup vs baseline: 3.0249x; 3.0249x over previous
import functools

import jax
import jax.numpy as jnp
from jax import lax
from jax.experimental import pallas as pl
from jax.experimental.pallas import tpu as pltpu
from jax.experimental.pallas import tpu_sc as plsc

VOCAB = 1000000
HIDDEN = 64
OUT = 2
B = 4096
L = 200

NC = 2
NS = 16
NW = NC * NS
SPW = B // NW
LANES = 16
NGRP = 13
LPAD = NGRP * LANES
G0 = 128
G1 = L - G0

VBLK = 8192
VGRID = -(-VOCAB // VBLK)


def _proj_body(w_ref, tabT_ref, p0_ref, p1_ref):
    p = jnp.dot(w_ref[...], tabT_ref[...], preferred_element_type=jnp.float32)
    p0_ref[...] = p[0]
    p1_ref[...] = p[1]


_proj = pl.pallas_call(
    _proj_body,
    grid=(VGRID,),
    in_specs=[
        pl.BlockSpec((OUT, HIDDEN), lambda i: (0, 0)),
        pl.BlockSpec((HIDDEN, VBLK), lambda i: (0, i)),
    ],
    out_specs=[
        pl.BlockSpec((VBLK,), lambda i: (i,)),
        pl.BlockSpec((VBLK,), lambda i: (i,)),
    ],
    out_shape=[
        jax.ShapeDtypeStruct((VOCAB,), jnp.float32),
        jax.ShapeDtypeStruct((VOCAB,), jnp.float32),
    ],
)


def _sc_body(text_ref, p0_ref, p1_ref, out_ref,
             idx_v, buf0, buf1, sums_v, sem0, sem1):
    c = lax.axis_index("c")
    s = lax.axis_index("s")
    wid = s * NC + c
    base = wid * SPW

    pltpu.sync_copy(text_ref.at[pl.ds(base, SPW)], idx_v)

    sems = (sem0, sem1)

    def fire(sample, parity):
        row = idx_v.at[sample]
        for pref, buf in ((p0_ref, buf0), (p1_ref, buf1)):
            pltpu.async_copy(pref.at[row.at[pl.ds(0, G0)]],
                             buf.at[parity].at[pl.ds(0, G0)], sems[parity])
            pltpu.async_copy(pref.at[row.at[pl.ds(G0, G1)]],
                             buf.at[parity].at[pl.ds(G0, G1)], sems[parity])

    def drain(parity):
        for buf in (buf0, buf1):
            pltpu.make_async_copy(p0_ref.at[idx_v.at[0].at[pl.ds(0, G0)]],
                                  buf.at[parity].at[pl.ds(0, G0)],
                                  sems[parity]).wait()
            pltpu.make_async_copy(p0_ref.at[idx_v.at[0].at[pl.ds(G0, G1)]],
                                  buf.at[parity].at[pl.ds(G0, G1)],
                                  sems[parity]).wait()

    fire(0, 0)
    fire(1, 1)

    lane = lax.iota(jnp.int32, LANES)
    tail = lane < (L - (NGRP - 1) * LANES)

    def pair_body(p, carry):
        for parity in range(2):
            sample = 2 * p + parity
            drain(parity)

            acc0 = jnp.zeros((LANES,), jnp.float32)
            acc1 = jnp.zeros((LANES,), jnp.float32)
            for g in range(NGRP - 1):
                acc0 = acc0 + buf0[parity, pl.ds(g * LANES, LANES)]
                acc1 = acc1 + buf1[parity, pl.ds(g * LANES, LANES)]
            g = NGRP - 1
            acc0 = acc0 + jnp.where(
                tail, buf0[parity, pl.ds(g * LANES, LANES)], 0.0)
            acc1 = acc1 + jnp.where(
                tail, buf1[parity, pl.ds(g * LANES, LANES)], 0.0)

            nxt = sample + 2

            @pl.when(nxt < SPW)
            def _():
                fire(nxt, parity)

            sums_v[sample, pl.ds(0, LANES)] = acc0
            sums_v[sample, pl.ds(LANES, LANES)] = acc1
        return carry

    lax.fori_loop(0, SPW // 2, pair_body, 0)

    pltpu.sync_copy(sums_v, out_ref.at[pl.ds(base, SPW)])


@functools.partial(
    pl.kernel,
    out_type=jax.ShapeDtypeStruct((B, 2 * LANES), jnp.float32),
    mesh=plsc.VectorSubcoreMesh(core_axis_name="c", subcore_axis_name="s",
                                num_cores=NC, num_subcores=NS),
    scratch_types=[
        pltpu.VMEM((SPW, L), jnp.int32),
        pltpu.VMEM((2, LPAD), jnp.float32),
        pltpu.VMEM((2, LPAD), jnp.float32),
        pltpu.VMEM((SPW, 2 * LANES), jnp.float32),
        pltpu.SemaphoreType.DMA,
        pltpu.SemaphoreType.DMA,
    ],
    compiler_params=pltpu.CompilerParams(use_tc_tiling_on_sc=False),
)
def _sc_sums(text_ref, p0_ref, p1_ref, out_ref,
             idx_v, buf0, buf1, sums_v, sem0, sem1):
    _sc_body(text_ref, p0_ref, p1_ref, out_ref,
             idx_v, buf0, buf1, sums_v, sem0, sem1)


def _finish_body(sums_ref, b_ref, o_ref):
    x = sums_ref[...] * (1.0 / L)
    s0 = jnp.sum(x[:, :LANES], axis=1, keepdims=True)
    s1 = jnp.sum(x[:, LANES:], axis=1, keepdims=True)
    o_ref[...] = jnp.concatenate([s0, s1], axis=1) + b_ref[...][None, :]


_finish = pl.pallas_call(
    _finish_body,
    out_shape=jax.ShapeDtypeStruct((B, OUT), jnp.float32),
)


def kernel(text, emb_table, fc1_w, fc1_b):
    tabT = emb_table.T
    p0, p1 = _proj(fc1_w, tabT)
    sums = _sc_sums(text.astype(jnp.int32), p0, p1)
    return _finish(sums, fc1_b)


# speedup vs baseline: 4.0013x; 1.3228x over previous
import functools

import jax
import jax.numpy as jnp
from jax import lax
from jax.experimental import pallas as pl
from jax.experimental.pallas import tpu as pltpu
from jax.experimental.pallas import tpu_sc as plsc

VOCAB = 1000000
HIDDEN = 64
OUT = 2
B = 4096
L = 200

NC = 2
NS = 16
NW = NC * NS
SPW = B // NW
LANES = 16
NGRP = 13
LPAD = NGRP * LANES
G0 = 128
G1 = L - G0

VBLK = 16384
VGRID = -(-VOCAB // VBLK)


def _proj_body(w_ref, tabT_ref, q_ref):
    p = jnp.dot(w_ref[...], tabT_ref[...], preferred_element_type=jnp.float32)
    lo = lax.bitcast_convert_type(p[0].astype(jnp.bfloat16), jnp.uint16)
    hi = lax.bitcast_convert_type(p[1].astype(jnp.bfloat16), jnp.uint16)
    word = hi.astype(jnp.uint32) << 16 | lo.astype(jnp.uint32)
    q_ref[...] = lax.bitcast_convert_type(word, jnp.int32)


_proj = pl.pallas_call(
    _proj_body,
    grid=(VGRID,),
    in_specs=[
        pl.BlockSpec((OUT, HIDDEN), lambda i: (0, 0)),
        pl.BlockSpec((HIDDEN, VBLK), lambda i: (0, i)),
    ],
    out_specs=pl.BlockSpec((VBLK,), lambda i: (i,)),
    out_shape=jax.ShapeDtypeStruct((VOCAB,), jnp.int32),
)


def _sc_body(text_ref, q_ref, out_ref, idx_v, buf, sums_v, sem0, sem1):
    c = lax.axis_index("c")
    s = lax.axis_index("s")
    wid = s * NC + c
    base = wid * SPW

    pltpu.sync_copy(text_ref.at[pl.ds(base, SPW)], idx_v)

    sems = (sem0, sem1)

    def fire(sample, parity):
        row = idx_v.at[sample]
        pltpu.async_copy(q_ref.at[row.at[pl.ds(0, G0)]],
                         buf.at[parity].at[pl.ds(0, G0)], sems[parity])
        pltpu.async_copy(q_ref.at[row.at[pl.ds(G0, G1)]],
                         buf.at[parity].at[pl.ds(G0, G1)], sems[parity])

    def drain(parity):
        pltpu.make_async_copy(q_ref.at[idx_v.at[0].at[pl.ds(0, G0)]],
                              buf.at[parity].at[pl.ds(0, G0)],
                              sems[parity]).wait()
        pltpu.make_async_copy(q_ref.at[idx_v.at[0].at[pl.ds(G0, G1)]],
                              buf.at[parity].at[pl.ds(G0, G1)],
                              sems[parity]).wait()

    fire(0, 0)
    fire(1, 1)

    lane = lax.iota(jnp.int32, LANES)
    tail = lane < (L - (NGRP - 1) * LANES)

    def pair_body(p, carry):
        for parity in range(2):
            sample = 2 * p + parity
            drain(parity)

            acc0 = jnp.zeros((LANES,), jnp.float32)
            acc1 = jnp.zeros((LANES,), jnp.float32)
            for g in range(NGRP):
                w32 = buf[parity, pl.ds(g * LANES, LANES)]
                pair = plsc.bitcast(w32, jnp.bfloat16)
                a, b = plsc.unpack(pair, format=plsc.PackFormat.INTERLEAVED)
                if g == NGRP - 1:
                    a = jnp.where(tail, a, 0.0)
                    b = jnp.where(tail, b, 0.0)
                acc0 = acc0 + a
                acc1 = acc1 + b

            nxt = sample + 2

            @pl.when(nxt < SPW)
            def _():
                fire(nxt, parity)

            sums_v[sample, pl.ds(0, LANES)] = acc0
            sums_v[sample, pl.ds(LANES, LANES)] = acc1
        return carry

    lax.fori_loop(0, SPW // 2, pair_body, 0)

    pltpu.sync_copy(sums_v, out_ref.at[pl.ds(base, SPW)])


@functools.partial(
    pl.kernel,
    out_type=jax.ShapeDtypeStruct((B, 2 * LANES), jnp.float32),
    mesh=plsc.VectorSubcoreMesh(core_axis_name="c", subcore_axis_name="s",
                                num_cores=NC, num_subcores=NS),
    scratch_types=[
        pltpu.VMEM((SPW, L), jnp.int32),
        pltpu.VMEM((2, LPAD), jnp.int32),
        pltpu.VMEM((SPW, 2 * LANES), jnp.float32),
        pltpu.SemaphoreType.DMA,
        pltpu.SemaphoreType.DMA,
    ],
    compiler_params=pltpu.CompilerParams(use_tc_tiling_on_sc=False,
                                         needs_layout_passes=False),
)
def _sc_sums(text_ref, q_ref, out_ref, idx_v, buf, sums_v, sem0, sem1):
    _sc_body(text_ref, q_ref, out_ref, idx_v, buf, sums_v, sem0, sem1)


def _finish_body(sums_ref, b_ref, o_ref):
    i = lax.broadcasted_iota(jnp.int32, (2 * LANES, OUT), 0)
    j = lax.broadcasted_iota(jnp.int32, (2 * LANES, OUT), 1)
    m = jnp.where(i // LANES == j, 1.0 / L, 0.0)
    o_ref[...] = (jnp.dot(sums_ref[...], m,
                          preferred_element_type=jnp.float32)
                  + b_ref[...][None, :])


_finish = pl.pallas_call(
    _finish_body,
    out_shape=jax.ShapeDtypeStruct((B, OUT), jnp.float32),
)


def kernel(text, emb_table, fc1_w, fc1_b):
    tabT = emb_table.T
    q = _proj(fc1_w, tabT)
    sums = _sc_sums(text.astype(jnp.int32), q)
    return _finish(sums, fc1_b)


# speedup vs baseline: 4.3557x; 1.0886x over previous
import functools

import jax
import jax.numpy as jnp
from jax import lax
from jax.experimental import pallas as pl
from jax.experimental.pallas import tpu as pltpu
from jax.experimental.pallas import tpu_sc as plsc

VOCAB = 1000000
HIDDEN = 64
OUT = 2
B = 4096
L = 200

NC = 2
NS = 16
NW = NC * NS
SPW = B // NW
LANES = 16
NGRP = 13
LPAD = NGRP * LANES
G0 = 128
G1 = L - G0

VBLK = 32768
VGRID = -(-VOCAB // VBLK)


def _proj_body(w_ref, tabT_ref, q_ref):
    p = jnp.dot(w_ref[...], tabT_ref[...],
                preferred_element_type=jnp.float32) * (1.0 / L)
    lo = lax.bitcast_convert_type(p[0].astype(jnp.bfloat16), jnp.uint16)
    hi = lax.bitcast_convert_type(p[1].astype(jnp.bfloat16), jnp.uint16)
    word = hi.astype(jnp.uint32) << 16 | lo.astype(jnp.uint32)
    q_ref[...] = lax.bitcast_convert_type(word, jnp.int32)


_proj = pl.pallas_call(
    _proj_body,
    grid=(VGRID,),
    in_specs=[
        pl.BlockSpec((OUT, HIDDEN), lambda i: (0, 0)),
        pl.BlockSpec((HIDDEN, VBLK), lambda i: (0, i)),
    ],
    out_specs=pl.BlockSpec((VBLK,), lambda i: (i,)),
    out_shape=jax.ShapeDtypeStruct((VOCAB,), jnp.int32),
)


def _sc_body(text_ref, q_ref, b_ref, out_ref, idx_v, buf, b_v, out_v,
             sem0, sem1):
    c = lax.axis_index("c")
    s = lax.axis_index("s")
    wid = s * NC + c
    base = wid * SPW

    pltpu.sync_copy(text_ref.at[pl.ds(base, SPW)], idx_v)
    pltpu.sync_copy(b_ref, b_v)

    sems = (sem0, sem1)

    def fire(sample, parity):
        row = idx_v.at[sample]
        pltpu.async_copy(q_ref.at[row.at[pl.ds(0, G0)]],
                         buf.at[parity].at[pl.ds(0, G0)], sems[parity])
        pltpu.async_copy(q_ref.at[row.at[pl.ds(G0, G1)]],
                         buf.at[parity].at[pl.ds(G0, G1)], sems[parity])

    def drain(parity):
        pltpu.make_async_copy(q_ref.at[idx_v.at[0].at[pl.ds(0, G0)]],
                              buf.at[parity].at[pl.ds(0, G0)],
                              sems[parity]).wait()
        pltpu.make_async_copy(q_ref.at[idx_v.at[0].at[pl.ds(G0, G1)]],
                              buf.at[parity].at[pl.ds(G0, G1)],
                              sems[parity]).wait()

    fire(0, 0)
    fire(1, 1)

    lane = lax.iota(jnp.int32, LANES)
    tail = lane < (L - (NGRP - 1) * LANES)
    last = lane == (LANES - 1)
    bvec = b_v[...]
    bias0 = jnp.where(lane == 0, bvec, 0.0)
    bias1 = jnp.where(lane == 1, bvec, 0.0)

    def pair_body(p, carry):
        for parity in range(2):
            sample = 2 * p + parity
            drain(parity)

            acc0 = jnp.zeros((LANES,), jnp.float32)
            acc1 = jnp.zeros((LANES,), jnp.float32)
            for g in range(NGRP):
                w32 = buf[parity, pl.ds(g * LANES, LANES)]
                pair = plsc.bitcast(w32, jnp.bfloat16)
                a, b = plsc.unpack(pair, format=plsc.PackFormat.INTERLEAVED)
                if g == NGRP - 1:
                    a = jnp.where(tail, a, 0.0)
                    b = jnp.where(tail, b, 0.0)
                acc0 = acc0 + a
                acc1 = acc1 + b

            nxt = sample + 2

            @pl.when(nxt < SPW)
            def _():
                fire(nxt, parity)

            c0 = plsc.cumsum(acc0 + bias0)
            c1 = plsc.cumsum(acc1 + bias1)
            pos = jnp.zeros((LANES,), jnp.int32) + OUT * sample
            plsc.store_scatter(out_v, [pos], c0, mask=last)
            plsc.store_scatter(out_v, [pos + 1], c1, mask=last)
        return carry

    lax.fori_loop(0, SPW // 2, pair_body, 0)

    pltpu.sync_copy(out_v, out_ref.at[pl.ds(OUT * base, OUT * SPW)])


@functools.partial(
    pl.kernel,
    out_type=jax.ShapeDtypeStruct((B * OUT,), jnp.float32),
    mesh=plsc.VectorSubcoreMesh(core_axis_name="c", subcore_axis_name="s",
                                num_cores=NC, num_subcores=NS),
    scratch_types=[
        pltpu.VMEM((SPW, L), jnp.int32),
        pltpu.VMEM((2, LPAD), jnp.int32),
        pltpu.VMEM((LANES,), jnp.float32),
        pltpu.VMEM((OUT * SPW,), jnp.float32),
        pltpu.SemaphoreType.DMA,
        pltpu.SemaphoreType.DMA,
    ],
    compiler_params=pltpu.CompilerParams(use_tc_tiling_on_sc=False,
                                         needs_layout_passes=False),
)
def _sc_sums(text_ref, q_ref, b_ref, out_ref, idx_v, buf, b_v, out_v,
             sem0, sem1):
    _sc_body(text_ref, q_ref, b_ref, out_ref, idx_v, buf, b_v, out_v,
             sem0, sem1)


def kernel(text, emb_table, fc1_w, fc1_b):
    tabT = emb_table.T
    q = _proj(fc1_w, tabT)
    b16 = jnp.zeros((LANES,), jnp.float32).at[:OUT].set(fc1_b)
    out = _sc_sums(text.astype(jnp.int32), q, b16)
    return out.reshape(B, OUT)


# speedup vs baseline: 5.6707x; 1.3019x over previous
import functools

import jax
import jax.numpy as jnp
from jax import lax
from jax.experimental import pallas as pl
from jax.experimental.pallas import tpu as pltpu
from jax.experimental.pallas import tpu_sc as plsc

VOCAB = 1000000
HIDDEN = 64
OUT = 2
B = 4096
L = 200

NC = 2
NS = 16
NW = NC * NS
SPW = B // NW
LANES = 16
NGRP = 13
LPAD = NGRP * LANES
G0 = 128
G1 = L - G0

VBLK = 32768
VGRID = -(-VOCAB // VBLK)
QPAD = VGRID * VBLK
CHUNK = QPAD // NS


def _proj_body(w_ref, tabT_ref, q_ref):
    p = jnp.dot(w_ref[...], tabT_ref[...],
                preferred_element_type=jnp.float32) * (1.0 / L)
    lo = lax.bitcast_convert_type(p[0].astype(jnp.bfloat16), jnp.uint16)
    hi = lax.bitcast_convert_type(p[1].astype(jnp.bfloat16), jnp.uint16)
    word = hi.astype(jnp.uint32) << 16 | lo.astype(jnp.uint32)
    q_ref[...] = lax.bitcast_convert_type(word, jnp.int32)


_proj = pl.pallas_call(
    _proj_body,
    grid=(VGRID,),
    in_specs=[
        pl.BlockSpec((OUT, HIDDEN), lambda i: (0, 0)),
        pl.BlockSpec((HIDDEN, VBLK), lambda i: (0, i)),
    ],
    out_specs=pl.BlockSpec((VBLK,), lambda i: (i,)),
    out_shape=jax.ShapeDtypeStruct((QPAD,), jnp.int32),
)


def _sc_body(text_ref, q_ref, b_ref, out_ref, idx_v, buf, b_v, out_v,
             qs_v, sem0, sem1):
    c = lax.axis_index("c")
    s = lax.axis_index("s")
    wid = s * NC + c
    base = wid * SPW

    pltpu.sync_copy(q_ref.at[pl.ds(s * CHUNK, CHUNK)],
                    qs_v.at[pl.ds(s * CHUNK, CHUNK)])
    pltpu.sync_copy(text_ref.at[pl.ds(base, SPW)], idx_v)
    pltpu.sync_copy(b_ref, b_v)
    plsc.subcore_barrier()

    sems = (sem0, sem1)

    def fire(sample, parity):
        row = idx_v.at[sample]
        pltpu.async_copy(qs_v.at[row.at[pl.ds(0, G0)]],
                         buf.at[parity].at[pl.ds(0, G0)], sems[parity])
        pltpu.async_copy(qs_v.at[row.at[pl.ds(G0, G1)]],
                         buf.at[parity].at[pl.ds(G0, G1)], sems[parity])

    def drain(parity):
        pltpu.make_async_copy(qs_v.at[idx_v.at[0].at[pl.ds(0, G0)]],
                              buf.at[parity].at[pl.ds(0, G0)],
                              sems[parity]).wait()
        pltpu.make_async_copy(qs_v.at[idx_v.at[0].at[pl.ds(G0, G1)]],
                              buf.at[parity].at[pl.ds(G0, G1)],
                              sems[parity]).wait()

    fire(0, 0)
    fire(1, 1)

    lane = lax.iota(jnp.int32, LANES)
    tail = lane < (L - (NGRP - 1) * LANES)
    last = lane == (LANES - 1)
    bvec = b_v[...]
    bias0 = jnp.where(lane == 0, bvec, 0.0)
    bias1 = jnp.where(lane == 1, bvec, 0.0)

    def pair_body(p, carry):
        for parity in range(2):
            sample = 2 * p + parity
            drain(parity)

            acc0 = jnp.zeros((LANES,), jnp.float32)
            acc1 = jnp.zeros((LANES,), jnp.float32)
            for g in range(NGRP):
                w32 = buf[parity, pl.ds(g * LANES, LANES)]
                pair = plsc.bitcast(w32, jnp.bfloat16)
                a, b = plsc.unpack(pair, format=plsc.PackFormat.INTERLEAVED)
                if g == NGRP - 1:
                    a = jnp.where(tail, a, 0.0)
                    b = jnp.where(tail, b, 0.0)
                acc0 = acc0 + a
                acc1 = acc1 + b

            nxt = sample + 2

            @pl.when(nxt < SPW)
            def _():
                fire(nxt, parity)

            c0 = plsc.cumsum(acc0 + bias0)
            c1 = plsc.cumsum(acc1 + bias1)
            pos = jnp.zeros((LANES,), jnp.int32) + OUT * sample
            plsc.store_scatter(out_v, [pos], c0, mask=last)
            plsc.store_scatter(out_v, [pos + 1], c1, mask=last)
        return carry

    lax.fori_loop(0, SPW // 2, pair_body, 0)

    pltpu.sync_copy(out_v, out_ref.at[pl.ds(OUT * base, OUT * SPW)])


@functools.partial(
    pl.kernel,
    out_type=jax.ShapeDtypeStruct((B * OUT,), jnp.float32),
    mesh=plsc.VectorSubcoreMesh(core_axis_name="c", subcore_axis_name="s",
                                num_cores=NC, num_subcores=NS),
    scratch_types=[
        pltpu.VMEM((SPW, L), jnp.int32),
        pltpu.VMEM((2, LPAD), jnp.int32),
        pltpu.VMEM((LANES,), jnp.float32),
        pltpu.VMEM((OUT * SPW,), jnp.float32),
        pltpu.VMEM_SHARED((QPAD,), jnp.int32),
        pltpu.SemaphoreType.DMA,
        pltpu.SemaphoreType.DMA,
    ],
    compiler_params=pltpu.CompilerParams(use_tc_tiling_on_sc=False,
                                         needs_layout_passes=False),
)
def _sc_sums(text_ref, q_ref, b_ref, out_ref, idx_v, buf, b_v, out_v,
             qs_v, sem0, sem1):
    _sc_body(text_ref, q_ref, b_ref, out_ref, idx_v, buf, b_v, out_v,
             qs_v, sem0, sem1)


def kernel(text, emb_table, fc1_w, fc1_b):
    tabT = emb_table.T
    q = _proj(fc1_w, tabT)
    b16 = jnp.zeros((LANES,), jnp.float32).at[:OUT].set(fc1_b)
    out = _sc_sums(text.astype(jnp.int32), q, b16)
    return out.reshape(B, OUT)
